# baseline (speedup 1.0000x reference)
import jax
import jax.numpy as jnp
from jax import lax
from jax.experimental import pallas as pl
from jax.experimental.pallas import tpu as pltpu
from jax.experimental.pallas import tpu_sc as plsc

_B, _F, _V = 16384, 26, 100000
_DE, _DH = 16, 32
_N = _B * _F
_NW = 32
_NPW = _N // _NW
_CH = 1024
_NCH = _NPW // _CH


def _ser_body(fidx_hbm, easy_hbm, hard_hbm, easy_out, hard_out,
              idx_v, easy_b, hard_b, sem_g):
    wid = lax.axis_index("s") * 2 + lax.axis_index("c")
    base = wid * _NPW

    pltpu.sync_copy(fidx_hbm.at[pl.ds(base, _NPW)], idx_v)

    def step(c, carry):
        sl = pl.ds(c * _CH, _CH)
        ce = pltpu.async_copy(easy_hbm.at[idx_v.at[sl]], easy_b, sem_g)
        ch = pltpu.async_copy(hard_hbm.at[idx_v.at[sl]], hard_b, sem_g)
        ce.wait()
        ch.wait()
        pltpu.sync_copy(easy_b, easy_out.at[pl.ds(base + c * _CH, _CH)])
        pltpu.sync_copy(hard_b, hard_out.at[pl.ds(base + c * _CH, _CH)])
        return carry

    lax.fori_loop(0, _NCH, step, 0)


@jax.jit
def _ser(fidx, easy_flat, hard_flat):
    mesh = plsc.VectorSubcoreMesh(core_axis_name="c", subcore_axis_name="s")
    return pl.kernel(
        _ser_body,
        out_type=(
            jax.ShapeDtypeStruct((_N, _DE), jnp.float32),
            jax.ShapeDtypeStruct((_N, _DH), jnp.float32),
        ),
        mesh=mesh,
        scratch_types=[
            pltpu.VMEM((_NPW,), jnp.int32),
            pltpu.VMEM((_CH, _DE), jnp.float32),
            pltpu.VMEM((_CH, _DH), jnp.float32),
            pltpu.SemaphoreType.DMA,
        ],
        compiler_params=pltpu.CompilerParams(use_tc_tiling_on_sc=False),
    )(fidx, easy_flat, hard_flat)


def kernel(X, easy_table, hard_table):
    fidx = (X + jnp.arange(_F, dtype=jnp.int32)[None, :] * _V).reshape(_N)
    easy_flat = easy_table.reshape(_F * _V, _DE)
    hard_flat = hard_table.reshape(_F * _V, _DH)
    easy_rows, hard_rows = _ser(fidx, easy_flat, hard_flat)
    return (easy_rows.reshape(_B, _F * _DE), hard_rows.reshape(_B, _F * _DH))


# speedup vs baseline: 1.2984x; 1.2984x over previous
import jax
import jax.numpy as jnp
from jax import lax
from jax.experimental import pallas as pl
from jax.experimental.pallas import tpu as pltpu
from jax.experimental.pallas import tpu_sc as plsc

_B, _F, _V = 16384, 26, 100000
_DE, _DH = 16, 32
_RE = _F * _DE
_RH = _F * _DH
_NW = 32
_EPW = _RE // _NW
_HPW = _RH // _NW
_CH = 4096
_NCH = _B // _CH


def _ser_body(xt_hbm, easy_hbm, hard_hbm, oute_hbm, outh_hbm,
              idx_v, buf_v, sem):
    wid = lax.axis_index("s") * 2 + lax.axis_index("c")

    def do_row(r, table_hbm, out_hbm, dlog):
        f = r // dlog
        pltpu.sync_copy(xt_hbm.at[f], idx_v)

        def chunk(c, carry):
            sl = pl.ds(c * _CH, _CH)
            pltpu.async_copy(table_hbm.at[r].at[idx_v.at[sl]],
                             buf_v, sem).wait()
            pltpu.sync_copy(buf_v, out_hbm.at[r, sl])
            return carry

        lax.fori_loop(0, _NCH, chunk, 0)

    def easy_row(t, carry):
        do_row(wid * _EPW + t, easy_hbm, oute_hbm, _DE)
        return carry

    lax.fori_loop(0, _EPW, easy_row, 0)

    def hard_row(t, carry):
        do_row(wid * _HPW + t, hard_hbm, outh_hbm, _DH)
        return carry

    lax.fori_loop(0, _HPW, hard_row, 0)


@jax.jit
def _ser(xt, easy_t, hard_t):
    mesh = plsc.VectorSubcoreMesh(core_axis_name="c", subcore_axis_name="s")
    return pl.kernel(
        _ser_body,
        out_type=(
            jax.ShapeDtypeStruct((_RE, _B), jnp.float32),
            jax.ShapeDtypeStruct((_RH, _B), jnp.float32),
        ),
        mesh=mesh,
        scratch_types=[
            pltpu.VMEM((_B,), jnp.int32),
            pltpu.VMEM((_CH,), jnp.float32),
            pltpu.SemaphoreType.DMA,
        ],
        compiler_params=pltpu.CompilerParams(use_tc_tiling_on_sc=False),
    )(xt, easy_t, hard_t)


def kernel(X, easy_table, hard_table):
    xt = X.T
    easy_t = jnp.transpose(easy_table, (0, 2, 1)).reshape(_RE, _V)
    hard_t = jnp.transpose(hard_table, (0, 2, 1)).reshape(_RH, _V)
    oute_t, outh_t = _ser(xt, easy_t, hard_t)
    return (oute_t.T.reshape(_B, _RE), outh_t.T.reshape(_B, _RH))


# speedup vs baseline: 1.3169x; 1.0142x over previous
import jax
import jax.numpy as jnp
from jax import lax
from jax.experimental import pallas as pl
from jax.experimental.pallas import tpu as pltpu
from jax.experimental.pallas import tpu_sc as plsc

_B, _F, _V = 16384, 26, 100000
_DE, _DH = 16, 32
_RE = _F * _DE
_RH = _F * _DH
_NW = 32
_EPW = _RE // _NW
_HPW = _RH // _NW
_CH = 4096
_NCH = _B // _CH


def _ser_body(xt_hbm, easy_hbm, hard_hbm, oute_hbm, outh_hbm,
              idx_v, buf_v, sem, sem_o):
    wid = lax.axis_index("s") * 2 + lax.axis_index("c")

    def do_row(r, table_hbm, out_hbm, dlog):
        f = r // dlog
        pltpu.sync_copy(xt_hbm.at[f], idx_v)

        outs = [None, None]
        for c in range(_NCH):
            b = c % 2
            sl = pl.ds(c * _CH, _CH)
            if outs[b] is not None:
                outs[b].wait()
            pltpu.async_copy(table_hbm.at[r].at[idx_v.at[sl]],
                             buf_v.at[b], sem).wait()
            cp = pltpu.async_copy(buf_v.at[b], out_hbm.at[r, sl], sem_o)
            outs[b] = cp
        for cp in outs:
            cp.wait()

    def easy_row(t, carry):
        do_row(wid * _EPW + t, easy_hbm, oute_hbm, _DE)
        return carry

    lax.fori_loop(0, _EPW, easy_row, 0)

    def hard_row(t, carry):
        do_row(wid * _HPW + t, hard_hbm, outh_hbm, _DH)
        return carry

    lax.fori_loop(0, _HPW, hard_row, 0)


@jax.jit
def _ser(xt, easy_t, hard_t):
    mesh = plsc.VectorSubcoreMesh(core_axis_name="c", subcore_axis_name="s")
    return pl.kernel(
        _ser_body,
        out_type=(
            jax.ShapeDtypeStruct((_RE, _B), jnp.float32),
            jax.ShapeDtypeStruct((_RH, _B), jnp.float32),
        ),
        mesh=mesh,
        scratch_types=[
            pltpu.VMEM((_B,), jnp.int32),
            pltpu.VMEM((2, _CH), jnp.float32),
            pltpu.SemaphoreType.DMA,
            pltpu.SemaphoreType.DMA,
        ],
        compiler_params=pltpu.CompilerParams(use_tc_tiling_on_sc=False),
    )(xt, easy_t, hard_t)


def kernel(X, easy_table, hard_table):
    xt = X.T
    easy_t = jnp.transpose(easy_table, (0, 2, 1)).reshape(_RE, _V)
    hard_t = jnp.transpose(hard_table, (0, 2, 1)).reshape(_RH, _V)
    oute_t, outh_t = _ser(xt, easy_t, hard_t)
    return (oute_t.T.reshape(_B, _RE), outh_t.T.reshape(_B, _RH))


# speedup vs baseline: 1.4884x; 1.1303x over previous
import jax
import jax.numpy as jnp
from jax import lax
from jax.experimental import pallas as pl
from jax.experimental.pallas import tpu as pltpu
from jax.experimental.pallas import tpu_sc as plsc

_B, _F, _V = 16384, 26, 100000
_DE, _DH = 16, 32
_RE = _F * _DE
_RH = _F * _DH
_NW = 32
_EPW = _RE // _NW
_HPW = _RH // _NW
_CH = 4096
_NCH = _B // _CH


def _ser_body(xt_hbm, table_hbm, out_hbm, n_per_w, dlog,
              idx_v, buf_v, sem, sem_o):
    wid = lax.axis_index("s") * 2 + lax.axis_index("c")

    def do_row(r):
        f = r // dlog
        pltpu.sync_copy(xt_hbm.at[f], idx_v)
        table_ref, out_ref = table_hbm, out_hbm

        outs = [None, None]
        for c in range(_NCH):
            b = c % 2
            sl = pl.ds(c * _CH, _CH)
            if outs[b] is not None:
                outs[b].wait()
            pltpu.async_copy(table_ref.at[r].at[idx_v.at[sl]],
                             buf_v.at[b], sem).wait()
            cp = pltpu.async_copy(buf_v.at[b], out_ref.at[r, sl], sem_o)
            outs[b] = cp
        for cp in outs:
            cp.wait()

    def row(t, carry):
        do_row(wid * n_per_w + t)
        return carry

    lax.fori_loop(0, n_per_w, row, 0)


import functools


@functools.partial(jax.jit, static_argnums=(2, 3, 4))
def _ser_one(xt, table_t, rows, n_per_w, dlog):
    mesh = plsc.VectorSubcoreMesh(core_axis_name="c", subcore_axis_name="s")
    body = functools.partial(_ser_body, n_per_w=n_per_w, dlog=dlog)

    def wrapped(xt_hbm, table_hbm, out_hbm, idx_v, buf_v, sem, sem_o):
        _ser_body(xt_hbm, table_hbm, out_hbm, n_per_w, dlog,
                  idx_v, buf_v, sem, sem_o)

    return pl.kernel(
        wrapped,
        out_type=jax.ShapeDtypeStruct((rows, _B), jnp.float32),
        mesh=mesh,
        scratch_types=[
            pltpu.VMEM((_B,), jnp.int32),
            pltpu.VMEM((2, _CH), jnp.float32),
            pltpu.SemaphoreType.DMA,
            pltpu.SemaphoreType.DMA,
        ],
        compiler_params=pltpu.CompilerParams(use_tc_tiling_on_sc=False),
    )(xt, table_t)


def kernel(X, easy_table, hard_table):
    xt = X.T
    easy_t = jnp.transpose(easy_table, (0, 2, 1)).reshape(_RE, _V)
    hard_t = jnp.transpose(hard_table, (0, 2, 1)).reshape(_RH, _V)
    oute_t = _ser_one(xt, easy_t, _RE, _EPW, _DE)
    outh_t = _ser_one(xt, hard_t, _RH, _HPW, _DH)
    return (oute_t.T.reshape(_B, _RE), outh_t.T.reshape(_B, _RH))
